# baseline (speedup 1.0000x reference)
import jax
import jax.numpy as jnp
from jax.experimental import pallas as pl
from jax.experimental.pallas import tpu as pltpu

N = 5000
IN = 512
H1 = 256
H2 = 64

_BM1 = 512
_BM = 512
_NB = 10
_NPAD = _NB * _BM
_NS = 4
_CS = _NPAD // _NS
_BTI = 1024
_BTJ = 2560
_LAST_VALID = N - (_NS - 1) * _CS

_DEF = jax.lax.Precision.DEFAULT


def _mask_strip(c, a):
    if c != _NS - 1:
        return a
    lane = jax.lax.broadcasted_iota(jnp.int32, a.shape, 1)
    return jnp.where(lane < _LAST_VALID, a, jnp.zeros((), a.dtype))


def _xw_body(x_ref, w0_ref, w1_ref, p0_ref, p1_ref):
    i = pl.program_id(0)
    rows = i * _BM1 + jax.lax.broadcasted_iota(jnp.int32, (_BM1, 1), 0)
    valid = rows < N
    xb = x_ref[...]
    z = jnp.zeros((), jnp.float32)
    p0_ref[...] = jnp.where(
        valid, jnp.dot(xb, w0_ref[...], preferred_element_type=jnp.float32,
                       precision=_DEF), z)
    p1_ref[...] = jnp.where(
        valid, jnp.dot(xb, w1_ref[...], preferred_element_type=jnp.float32,
                       precision=_DEF), z)


def _phase2_j(s):
    return 2 * _NB - 1 - s


def _enc_body(*refs):
    a0_refs = refs[0:_NS]
    a1_refs = refs[_NS:2 * _NS]
    p0_ref, p1_ref, w2_ref = refs[2 * _NS:2 * _NS + 3]
    emb_ref, embb_ref, embt_ref, mw2_ref = refs[2 * _NS + 3:]
    s = pl.program_id(0)

    @pl.when(s < _NB)
    def _phase1():
        acc0 = jnp.zeros((_BM, H1), jnp.float32)
        acc1 = jnp.zeros((_BM, H1), jnp.float32)
        for c in range(_NS):
            a0c = _mask_strip(c, a0_refs[c][0])
            a1c = _mask_strip(c, a1_refs[c][0])
            p0c = p0_ref[c * _CS:(c + 1) * _CS, :]
            p1c = p1_ref[c * _CS:(c + 1) * _CS, :]
            acc0 = acc0 + jnp.dot(a0c, p0c, preferred_element_type=jnp.float32,
                                  precision=_DEF)
            acc1 = acc1 + jnp.dot(a1c, p1c, preferred_element_type=jnp.float32,
                                  precision=_DEF)
        h0 = jnp.maximum(acc0, 0.0)
        h1 = jnp.maximum(acc1, 0.0)
        mean = (h0 + h1) * 0.5
        mv = jnp.dot(mean, w2_ref[...], preferred_element_type=jnp.float32,
                     precision=_DEF)
        rows = s * _BM + jax.lax.broadcasted_iota(jnp.int32, (_BM, 1), 0)
        mv = jnp.where(rows < N, mv, jnp.zeros((), jnp.float32))
        mw2_ref[pl.ds(s * _BM, _BM), :] = mv

    @pl.when(s >= _NB)
    def _phase2():
        e = jnp.zeros((_BM, H2), jnp.float32)
        for c in range(_NS):
            a0c = _mask_strip(c, a0_refs[c][0])
            mc = mw2_ref[c * _CS:(c + 1) * _CS, :]
            e = e + jnp.dot(a0c, mc, preferred_element_type=jnp.float32,
                            precision=_DEF)
        e = jnp.maximum(e, 0.0)
        emb_ref[...] = e
        eb = e.astype(jnp.bfloat16)
        embb_ref[...] = eb
        embt_ref[...] = eb.T


def _recon_body(ei_ref, etj_ref, out0_ref, out1_ref):
    r = jnp.dot(ei_ref[...], etj_ref[...], preferred_element_type=jnp.float32)
    out0_ref[...] = r
    out1_ref[...] = r


def _a0_idx(c):
    def idx(s):
        return (0, jnp.where(s < _NB, s, _phase2_j(s)), c)
    return idx


def _a1_idx(c):
    def idx(s):
        return (1, jnp.minimum(s, _NB - 1), c)
    return idx


def _emb_idx(s):
    return (jnp.where(s < _NB, 0, _phase2_j(s)), 0)


def kernel(x, adjs, W0, W1, W2):
    p0, p1 = pl.pallas_call(
        _xw_body,
        grid=(_NPAD // _BM1,),
        in_specs=[
            pl.BlockSpec((_BM1, IN), lambda i: (i, 0)),
            pl.BlockSpec((IN, H1), lambda i: (0, 0)),
            pl.BlockSpec((IN, H1), lambda i: (0, 0)),
        ],
        out_specs=[
            pl.BlockSpec((_BM1, H1), lambda i: (i, 0)),
            pl.BlockSpec((_BM1, H1), lambda i: (i, 0)),
        ],
        out_shape=[
            jax.ShapeDtypeStruct((_NPAD, H1), jnp.float32),
            jax.ShapeDtypeStruct((_NPAD, H1), jnp.float32),
        ],
    )(x, W0, W1)

    adj_specs = (
        [pl.BlockSpec((1, _BM, _CS), _a0_idx(c)) for c in range(_NS)]
        + [pl.BlockSpec((1, _BM, _CS), _a1_idx(c)) for c in range(_NS)]
    )
    emb, embb, embt = pl.pallas_call(
        _enc_body,
        grid=(2 * _NB,),
        in_specs=adj_specs + [
            pl.BlockSpec((_NPAD, H1), lambda s: (0, 0)),
            pl.BlockSpec((_NPAD, H1), lambda s: (0, 0)),
            pl.BlockSpec((H1, H2), lambda s: (0, 0)),
        ],
        out_specs=[
            pl.BlockSpec((_BM, H2), _emb_idx),
            pl.BlockSpec((_BM, H2), _emb_idx),
            pl.BlockSpec((H2, _BM), lambda s: (0, _emb_idx(s)[0])),
        ],
        out_shape=[
            jax.ShapeDtypeStruct((N, H2), jnp.float32),
            jax.ShapeDtypeStruct((N, H2), jnp.bfloat16),
            jax.ShapeDtypeStruct((H2, N), jnp.bfloat16),
        ],
        scratch_shapes=[
            pltpu.VMEM((_NPAD, H2), jnp.float32),
        ],
    )(*([adjs] * (2 * _NS)), p0, p1, W2)

    recon0, recon1 = pl.pallas_call(
        _recon_body,
        grid=(pl.cdiv(N, _BTI), pl.cdiv(N, _BTJ)),
        in_specs=[
            pl.BlockSpec((_BTI, H2), lambda i, j: (i, 0)),
            pl.BlockSpec((H2, _BTJ), lambda i, j: (0, j)),
        ],
        out_specs=[
            pl.BlockSpec((_BTI, _BTJ), lambda i, j: (i, j)),
            pl.BlockSpec((_BTI, _BTJ), lambda i, j: (i, j)),
        ],
        out_shape=[
            jax.ShapeDtypeStruct((N, N), jnp.float32),
            jax.ShapeDtypeStruct((N, N), jnp.float32),
        ],
    )(embb, embt)

    return emb, recon0, recon1


# speedup vs baseline: 1.0020x; 1.0020x over previous
import jax
import jax.numpy as jnp
from jax.experimental import pallas as pl
from jax.experimental.pallas import tpu as pltpu

N = 5000
IN = 512
H1 = 256
H2 = 64

_BM1 = 512
_BM = 320
_NB = 16
_NPAD = _NB * _BM
_RB = 6
_RES = _RB * _BM
_NS = 4
_CS = _NPAD // _NS
_BTI = 1024
_BTJ = 1280
_LAST_VALID = N - (_NS - 1) * _CS


def _mask_strip(c, a):
    if c != _NS - 1:
        return a
    lane = jax.lax.broadcasted_iota(jnp.int32, a.shape, 1)
    return jnp.where(lane < _LAST_VALID, a, jnp.zeros((), a.dtype))


def _xw_body(x_ref, w0_ref, w1_ref, p0_ref, p1_ref):
    i = pl.program_id(0)
    rows = i * _BM1 + jax.lax.broadcasted_iota(jnp.int32, (_BM1, 1), 0)
    valid = rows < N
    xb = x_ref[...].astype(jnp.bfloat16)
    w0 = w0_ref[...].astype(jnp.bfloat16)
    w1 = w1_ref[...].astype(jnp.bfloat16)
    z = jnp.zeros((), jnp.float32)
    p0 = jnp.where(valid, jnp.dot(xb, w0, preferred_element_type=jnp.float32), z)
    p1 = jnp.where(valid, jnp.dot(xb, w1, preferred_element_type=jnp.float32), z)
    p0_ref[...] = p0.astype(jnp.bfloat16)
    p1_ref[...] = p1.astype(jnp.bfloat16)


def _phase2_j(s):
    t = s - _NB
    return jnp.where(t < _RB, t, _NB - 1 - (t - _RB))


def _enc_body(*refs):
    a0_refs = refs[0:_NS]
    a1_refs = refs[_NS:2 * _NS]
    p0_ref, p1_ref, w2_ref = refs[2 * _NS:2 * _NS + 3]
    emb_ref, embb_ref, a0s_ref, mw2_ref = refs[2 * _NS + 3:]
    s = pl.program_id(0)

    @pl.when(s < _NB)
    def _phase1():
        acc0 = jnp.zeros((_BM, H1), jnp.float32)
        acc1 = jnp.zeros((_BM, H1), jnp.float32)
        for c in range(_NS):
            a0c = _mask_strip(c, a0_refs[c][0].astype(jnp.bfloat16))
            a1c = _mask_strip(c, a1_refs[c][0].astype(jnp.bfloat16))

            @pl.when(s < _RB)
            def _stash(a0c=a0c, c=c):
                a0s_ref[pl.ds(s * _BM, _BM), c * _CS:(c + 1) * _CS] = a0c

            p0c = p0_ref[c * _CS:(c + 1) * _CS, :]
            p1c = p1_ref[c * _CS:(c + 1) * _CS, :]
            acc0 = acc0 + jnp.dot(a0c, p0c, preferred_element_type=jnp.float32)
            acc1 = acc1 + jnp.dot(a1c, p1c, preferred_element_type=jnp.float32)
        h0 = jnp.maximum(acc0, 0.0)
        h1 = jnp.maximum(acc1, 0.0)
        mean = ((h0 + h1) * 0.5).astype(jnp.bfloat16)
        w2 = w2_ref[...].astype(jnp.bfloat16)
        mv = jnp.dot(mean, w2, preferred_element_type=jnp.float32)
        rows = s * _BM + jax.lax.broadcasted_iota(jnp.int32, (_BM, 1), 0)
        mv = jnp.where(rows < N, mv, jnp.zeros((), jnp.float32))
        mw2_ref[pl.ds(s * _BM, _BM), :] = mv.astype(jnp.bfloat16)

    @pl.when(s >= _NB)
    def _phase2():
        t = s - _NB
        j = _phase2_j(s)

        def emit(e):
            e = jnp.maximum(e, 0.0)
            emb_ref[...] = e
            embb_ref[...] = e.astype(jnp.bfloat16)

        @pl.when(t < _RB)
        def _resident():
            rows = a0s_ref[pl.ds(j * _BM, _BM), :]
            emit(jnp.dot(rows, mw2_ref[...], preferred_element_type=jnp.float32))

        @pl.when(t >= _RB)
        def _tail():
            e = jnp.zeros((_BM, H2), jnp.float32)
            for c in range(_NS):
                a0c = _mask_strip(c, a0_refs[c][0].astype(jnp.bfloat16))
                mc = mw2_ref[c * _CS:(c + 1) * _CS, :]
                e = e + jnp.dot(a0c, mc, preferred_element_type=jnp.float32)
            emit(e)


def _recon_body(ei_ref, ej_ref, out0_ref, out1_ref):
    r = jnp.dot(ei_ref[...], ej_ref[...].T, preferred_element_type=jnp.float32)
    out0_ref[...] = r
    out1_ref[...] = r


def _a0_idx(c):
    def idx(s):
        t = s - _NB
        tail = jnp.where(t < _RB, _NB - 1, _NB - 1 - (t - _RB))
        return (0, jnp.where(s < _NB, s, tail), c)
    return idx


def _a1_idx(c):
    def idx(s):
        return (1, jnp.minimum(s, _NB - 1), c)
    return idx


def _emb_idx(s):
    return (jnp.where(s < _NB, 0, _phase2_j(s)), 0)


def kernel(x, adjs, W0, W1, W2):
    p0, p1 = pl.pallas_call(
        _xw_body,
        grid=(_NPAD // _BM1,),
        in_specs=[
            pl.BlockSpec((_BM1, IN), lambda i: (i, 0)),
            pl.BlockSpec((IN, H1), lambda i: (0, 0)),
            pl.BlockSpec((IN, H1), lambda i: (0, 0)),
        ],
        out_specs=[
            pl.BlockSpec((_BM1, H1), lambda i: (i, 0)),
            pl.BlockSpec((_BM1, H1), lambda i: (i, 0)),
        ],
        out_shape=[
            jax.ShapeDtypeStruct((_NPAD, H1), jnp.bfloat16),
            jax.ShapeDtypeStruct((_NPAD, H1), jnp.bfloat16),
        ],
    )(x, W0, W1)

    adj_specs = (
        [pl.BlockSpec((1, _BM, _CS), _a0_idx(c)) for c in range(_NS)]
        + [pl.BlockSpec((1, _BM, _CS), _a1_idx(c)) for c in range(_NS)]
    )
    emb, embb = pl.pallas_call(
        _enc_body,
        grid=(2 * _NB,),
        in_specs=adj_specs + [
            pl.BlockSpec((_NPAD, H1), lambda s: (0, 0)),
            pl.BlockSpec((_NPAD, H1), lambda s: (0, 0)),
            pl.BlockSpec((H1, H2), lambda s: (0, 0)),
        ],
        out_specs=[
            pl.BlockSpec((_BM, H2), _emb_idx),
            pl.BlockSpec((_BM, H2), _emb_idx),
        ],
        out_shape=[
            jax.ShapeDtypeStruct((N, H2), jnp.float32),
            jax.ShapeDtypeStruct((N, H2), jnp.bfloat16),
        ],
        scratch_shapes=[
            pltpu.VMEM((_RES, _NPAD), jnp.bfloat16),
            pltpu.VMEM((_NPAD, H2), jnp.bfloat16),
        ],
    )(*([adjs] * (2 * _NS)), p0, p1, W2)

    recon0, recon1 = pl.pallas_call(
        _recon_body,
        grid=(pl.cdiv(N, _BTI), pl.cdiv(N, _BTJ)),
        in_specs=[
            pl.BlockSpec((_BTI, H2), lambda i, j: (i, 0)),
            pl.BlockSpec((_BTJ, H2), lambda i, j: (j, 0)),
        ],
        out_specs=[
            pl.BlockSpec((_BTI, _BTJ), lambda i, j: (i, j)),
            pl.BlockSpec((_BTI, _BTJ), lambda i, j: (i, j)),
        ],
        out_shape=[
            jax.ShapeDtypeStruct((N, N), jnp.float32),
            jax.ShapeDtypeStruct((N, N), jnp.float32),
        ],
    )(embb, embb)

    return emb, recon0, recon1


# speedup vs baseline: 1.0164x; 1.0144x over previous
import jax
import jax.numpy as jnp
from jax.experimental import pallas as pl
from jax.experimental.pallas import tpu as pltpu

N = 5000
IN = 512
H1 = 256
H2 = 64

_BM1 = 512
_BM = 256
_NB = 20
_NPAD = _NB * _BM
_NS = 4
_CS = _NPAD // _NS
_BTI = 1024
_BTJ = 1280
_LAST_VALID = N - (_NS - 1) * _CS


def _mask_strip(c, a):
    if c != _NS - 1:
        return a
    lane = jax.lax.broadcasted_iota(jnp.int32, a.shape, 1)
    return jnp.where(lane < _LAST_VALID, a, jnp.zeros((), a.dtype))


def _xw_body(x_ref, w0_ref, w1_ref, p0_ref, p1_ref):
    i = pl.program_id(0)
    rows = i * _BM1 + jax.lax.broadcasted_iota(jnp.int32, (_BM1, 1), 0)
    valid = rows < N
    xb = x_ref[...].astype(jnp.bfloat16)
    w0 = w0_ref[...].astype(jnp.bfloat16)
    w1 = w1_ref[...].astype(jnp.bfloat16)
    z = jnp.zeros((), jnp.float32)
    p0 = jnp.where(valid, jnp.dot(xb, w0, preferred_element_type=jnp.float32), z)
    p1 = jnp.where(valid, jnp.dot(xb, w1, preferred_element_type=jnp.float32), z)
    p0_ref[...] = p0.astype(jnp.bfloat16)
    p1_ref[...] = p1.astype(jnp.bfloat16)


def _enc_body(*refs):
    a0_refs = refs[0:_NS]
    a1_refs = refs[_NS:2 * _NS]
    col_ref, p0_ref, p1_ref, w2_ref = refs[2 * _NS:2 * _NS + 4]
    emb_ref, embb_ref, acc_ref = refs[2 * _NS + 4:]
    s = pl.program_id(0)

    @pl.when(s == 0)
    def _init():
        acc_ref[...] = jnp.zeros((N, H2), jnp.float32)

    acc0 = jnp.zeros((_BM, H1), jnp.float32)
    acc1 = jnp.zeros((_BM, H1), jnp.float32)
    for c in range(_NS):
        a0c = _mask_strip(c, a0_refs[c][0].astype(jnp.bfloat16))
        a1c = _mask_strip(c, a1_refs[c][0].astype(jnp.bfloat16))
        p0c = p0_ref[c * _CS:(c + 1) * _CS, :]
        p1c = p1_ref[c * _CS:(c + 1) * _CS, :]
        acc0 = acc0 + jnp.dot(a0c, p0c, preferred_element_type=jnp.float32)
        acc1 = acc1 + jnp.dot(a1c, p1c, preferred_element_type=jnp.float32)
    h0 = jnp.maximum(acc0, 0.0)
    h1 = jnp.maximum(acc1, 0.0)
    mean = (h0 + h1) * 0.5
    rows = s * _BM + jax.lax.broadcasted_iota(jnp.int32, (_BM, 1), 0)
    mean = jnp.where(rows < N, mean, jnp.zeros((), jnp.float32)).astype(jnp.bfloat16)
    w2 = w2_ref[...].astype(jnp.bfloat16)
    mk2 = jnp.dot(mean, w2, preferred_element_type=jnp.float32).astype(jnp.bfloat16)

    colp = col_ref[0].astype(jnp.bfloat16)
    lane = jax.lax.broadcasted_iota(jnp.int32, (N, _BM), 1)
    colp = jnp.where(lane < N - s * _BM, colp, jnp.zeros((), jnp.bfloat16))
    acc_ref[...] = acc_ref[...] + jnp.dot(colp, mk2, preferred_element_type=jnp.float32)

    @pl.when(s == _NB - 1)
    def _emit():
        e = jnp.maximum(acc_ref[...], 0.0)
        emb_ref[...] = e
        embb_ref[...] = e.astype(jnp.bfloat16)


def _recon_body(ei_ref, ej_ref, out0_ref, out1_ref):
    r = jnp.dot(ei_ref[...], ej_ref[...].T, preferred_element_type=jnp.float32)
    out0_ref[...] = r
    out1_ref[...] = r


def _a_idx(v, c):
    def idx(s):
        return (v, s, c)
    return idx


def kernel(x, adjs, W0, W1, W2):
    p0, p1 = pl.pallas_call(
        _xw_body,
        grid=(_NPAD // _BM1,),
        in_specs=[
            pl.BlockSpec((_BM1, IN), lambda i: (i, 0)),
            pl.BlockSpec((IN, H1), lambda i: (0, 0)),
            pl.BlockSpec((IN, H1), lambda i: (0, 0)),
        ],
        out_specs=[
            pl.BlockSpec((_BM1, H1), lambda i: (i, 0)),
            pl.BlockSpec((_BM1, H1), lambda i: (i, 0)),
        ],
        out_shape=[
            jax.ShapeDtypeStruct((_NPAD, H1), jnp.bfloat16),
            jax.ShapeDtypeStruct((_NPAD, H1), jnp.bfloat16),
        ],
    )(x, W0, W1)

    adj_specs = (
        [pl.BlockSpec((1, _BM, _CS), _a_idx(0, c)) for c in range(_NS)]
        + [pl.BlockSpec((1, _BM, _CS), _a_idx(1, c)) for c in range(_NS)]
        + [pl.BlockSpec((1, N, _BM), lambda s: (0, 0, s))]
    )
    emb, embb = pl.pallas_call(
        _enc_body,
        grid=(_NB,),
        in_specs=adj_specs + [
            pl.BlockSpec((_NPAD, H1), lambda s: (0, 0)),
            pl.BlockSpec((_NPAD, H1), lambda s: (0, 0)),
            pl.BlockSpec((H1, H2), lambda s: (0, 0)),
        ],
        out_specs=[
            pl.BlockSpec((N, H2), lambda s: (0, 0)),
            pl.BlockSpec((N, H2), lambda s: (0, 0)),
        ],
        out_shape=[
            jax.ShapeDtypeStruct((N, H2), jnp.float32),
            jax.ShapeDtypeStruct((N, H2), jnp.bfloat16),
        ],
        scratch_shapes=[
            pltpu.VMEM((N, H2), jnp.float32),
        ],
    )(*([adjs] * (2 * _NS + 1)), p0, p1, W2)

    recon0, recon1 = pl.pallas_call(
        _recon_body,
        grid=(pl.cdiv(N, _BTI), pl.cdiv(N, _BTJ)),
        in_specs=[
            pl.BlockSpec((_BTI, H2), lambda i, j: (i, 0)),
            pl.BlockSpec((_BTJ, H2), lambda i, j: (j, 0)),
        ],
        out_specs=[
            pl.BlockSpec((_BTI, _BTJ), lambda i, j: (i, j)),
            pl.BlockSpec((_BTI, _BTJ), lambda i, j: (i, j)),
        ],
        out_shape=[
            jax.ShapeDtypeStruct((N, N), jnp.float32),
            jax.ShapeDtypeStruct((N, N), jnp.float32),
        ],
    )(embb, embb)

    return emb, recon0, recon1
